# Initial kernel scaffold; baseline (speedup 1.0000x reference)
import jax
import jax.numpy as jnp
from jax.experimental import pallas as pl


def kernel(x, mapping, mem):
    raise NotImplementedError("write your pallas kernel here")


# speedup vs baseline: 1.4620x; 1.4620x over previous
import functools

import jax
import jax.numpy as jnp
from jax import lax
from jax.experimental import pallas as pl
from jax.experimental.pallas import tpu as pltpu
from jax.experimental.pallas import tpu_sc as plsc

INPUT_DIM = 12288
OUT_DIM = 16
NBITS = 12
N_NODES = INPUT_DIM // NBITS
N_ENTRIES = 2 ** NBITS
BATCH = 4096
LANES = 16
KEY_BLOCKS = N_NODES // LANES
IDX_MINOR = 128
N_CHUNKS = N_NODES // IDX_MINOR


def _make_kernel(num_workers):
  rows_per_w = BATCH // num_workers
  mesh = plsc.VectorSubcoreMesh(core_axis_name="c", subcore_axis_name="s")
  num_cores = mesh.num_cores

  @functools.partial(
      pl.kernel,
      out_type=jax.ShapeDtypeStruct((BATCH, OUT_DIM), jnp.float32),
      mesh=mesh,
      scratch_types=[
          pltpu.VMEM((INPUT_DIM,), jnp.int32),
          pltpu.VMEM((INPUT_DIM,), jnp.int32),
          pltpu.VMEM((N_CHUNKS, IDX_MINOR), jnp.int32),
          pltpu.VMEM((N_NODES, OUT_DIM), jnp.float32),
          pltpu.VMEM((rows_per_w, OUT_DIM), jnp.float32),
          pltpu.SemaphoreType.DMA,
      ],
      compiler_params=pltpu.CompilerParams(
          needs_layout_passes=False, use_tc_tiling_on_sc=False),
  )
  def k(x_hbm, perm_hbm, mem_hbm, out_hbm, perm_v, xrow_v, gidx_v, rows_v,
        out_v, sem):
    wid = lax.axis_index("s") * num_cores + lax.axis_index("c")
    base = wid * rows_per_w
    pltpu.sync_copy(perm_hbm, perm_v)
    lane = lax.broadcasted_iota(jnp.int32, (LANES,), 0)
    node_off = lane * N_ENTRIES

    def row_body(b, _):
      pltpu.sync_copy(x_hbm.at[base + b], xrow_v)

      def key_body_dyn(nb, _):
        key = jnp.zeros((LANES,), jnp.int32)
        for j in range(NBITS):
          idx = perm_v[pl.ds(j * N_NODES + nb * LANES, LANES)]
          bits = plsc.load_gather(xrow_v, [idx])
          key = key | (bits << j)
        gid = key + node_off + nb * (LANES * N_ENTRIES)
        row = nb // (IDX_MINOR // LANES)
        col = (nb - row * (IDX_MINOR // LANES)) * LANES
        gidx_v[row, pl.ds(col, LANES)] = gid
        return _

      lax.fori_loop(0, KEY_BLOCKS, key_body_dyn, 0, unroll=False)

      descs = []
      for c in range(N_CHUNKS):
        descs.append(
            pltpu.async_copy(
                mem_hbm.at[gidx_v.at[c]],
                rows_v.at[pl.ds(c * IDX_MINOR, IDX_MINOR)],
                sem,
            ))
      for d in descs:
        d.wait()

      def red_body(r, acc):
        for u in range(4):
          acc = acc + rows_v[r * 4 + u]
        return acc

      acc = lax.fori_loop(0, N_NODES // 4, red_body,
                          jnp.zeros((LANES,), jnp.float32), unroll=False)
      out_v[b] = acc * jnp.float32(1.0 / NBITS)
      return _

    lax.fori_loop(0, rows_per_w, row_body, 0, unroll=False)
    pltpu.sync_copy(out_v, out_hbm.at[pl.ds(base, rows_per_w)])

  return k


def kernel(x, mapping, mem):
  perm = mapping.reshape(N_NODES, NBITS).T.reshape(-1).astype(jnp.int32)
  mem2 = mem.reshape(N_NODES * N_ENTRIES, OUT_DIM)
  info = plsc.get_sparse_core_info()
  nw = info.num_cores * info.num_subcores
  k = _make_kernel(nw)
  return k(x, perm, mem2)


# speedup vs baseline: 1.7214x; 1.1775x over previous
import functools

import jax
import jax.numpy as jnp
from jax import lax
from jax.experimental import pallas as pl
from jax.experimental.pallas import tpu as pltpu
from jax.experimental.pallas import tpu_sc as plsc

INPUT_DIM = 12288
OUT_DIM = 16
NBITS = 12
N_NODES = INPUT_DIM // NBITS
N_ENTRIES = 2 ** NBITS
BATCH = 4096
LANES = 16
KEY_BLOCKS = N_NODES // LANES
IDX_MINOR = 128
N_CHUNKS = N_NODES // IDX_MINOR
RED_UNROLL = 8


def _make_kernel(num_workers):
  rows_per_w = BATCH // num_workers
  mesh = plsc.VectorSubcoreMesh(core_axis_name="c", subcore_axis_name="s")
  num_cores = mesh.num_cores

  @functools.partial(
      pl.kernel,
      out_type=jax.ShapeDtypeStruct((BATCH, OUT_DIM), jnp.float32),
      mesh=mesh,
      scratch_types=[
          pltpu.VMEM((INPUT_DIM,), jnp.int32),
          pltpu.VMEM((2, INPUT_DIM), jnp.int32),
          pltpu.VMEM((2, N_CHUNKS, IDX_MINOR), jnp.int32),
          pltpu.VMEM((2, N_NODES, OUT_DIM), jnp.float32),
          pltpu.VMEM((rows_per_w, OUT_DIM), jnp.float32),
          pltpu.SemaphoreType.DMA,
          pltpu.SemaphoreType.DMA,
          pltpu.SemaphoreType.DMA,
      ],
      compiler_params=pltpu.CompilerParams(
          needs_layout_passes=False, use_tc_tiling_on_sc=False),
  )
  def k(x_hbm, perm_hbm, mem_hbm, out_hbm, perm_v, xrow_v, gidx_v, rows_v,
        out_v, sem_x, sem_g0, sem_g1):
    sem_g = (sem_g0, sem_g1)
    wid = lax.axis_index("s") * num_cores + lax.axis_index("c")
    base = wid * rows_per_w
    pltpu.sync_copy(perm_hbm, perm_v)
    lane = lax.broadcasted_iota(jnp.int32, (LANES,), 0)
    node_off = lane * N_ENTRIES

    def compute_keys(b, s):
      xr = xrow_v.at[s]

      def key_body(nb, _):
        key = jnp.zeros((LANES,), jnp.int32)
        for j in range(NBITS):
          idx = perm_v[pl.ds(j * N_NODES + nb * LANES, LANES)]
          bits = plsc.load_gather(xr, [idx])
          key = key | (bits << j)
        gid = key + node_off + nb * (LANES * N_ENTRIES)
        row = nb // (IDX_MINOR // LANES)
        col = (nb - row * (IDX_MINOR // LANES)) * LANES
        gidx_v[s, row, pl.ds(col, LANES)] = gid
        return _

      lax.fori_loop(0, KEY_BLOCKS, key_body, 0, unroll=False)

    def fire_gathers(s):
      for c in range(N_CHUNKS):
        pltpu.async_copy(
            mem_hbm.at[gidx_v.at[s, c]],
            rows_v.at[s, pl.ds(c * IDX_MINOR, IDX_MINOR)],
            sem_g[s],
        )

    def drain_gathers(s):
      for c in range(N_CHUNKS):
        pltpu.make_async_copy(
            mem_hbm.at[gidx_v.at[s, c]],
            rows_v.at[s, pl.ds(c * IDX_MINOR, IDX_MINOR)],
            sem_g[s],
        ).wait()

    def reduce_row(b, s):
      def red_body(r, accs):
        a0, a1, a2, a3 = accs
        for u in range(RED_UNROLL):
          v = rows_v[s, r * RED_UNROLL + u]
          if u % 4 == 0:
            a0 = a0 + v
          elif u % 4 == 1:
            a1 = a1 + v
          elif u % 4 == 2:
            a2 = a2 + v
          else:
            a3 = a3 + v
        return (a0, a1, a2, a3)

      z = jnp.zeros((LANES,), jnp.float32)
      a0, a1, a2, a3 = lax.fori_loop(0, N_NODES // RED_UNROLL, red_body,
                                     (z, z, z, z), unroll=False)
      out_v[b] = ((a0 + a1) + (a2 + a3)) * jnp.float32(1.0 / NBITS)

    def issue_x(b, s):
      row = jnp.minimum(base + b, BATCH - 1)
      pltpu.async_copy(x_hbm.at[row], xrow_v.at[s], sem_x)

    def wait_x(b, s):
      row = jnp.minimum(base + b, BATCH - 1)
      pltpu.make_async_copy(x_hbm.at[row], xrow_v.at[s], sem_x).wait()

    issue_x(0, 0)
    wait_x(0, 0)
    compute_keys(0, 0)
    issue_x(1, 1)
    fire_gathers(0)

    def pair_body(g, _):
      for s in (1, 0):
        b = 2 * g + (s if s == 1 else 2)
        last = b >= rows_per_w

        @pl.when(jnp.logical_not(last))
        def _do():
          wait_x(b, s)
          compute_keys(b, s)

          @pl.when(b + 1 < rows_per_w)
          def _prefetch():
            issue_x(b + 1, 1 - s)

          fire_gathers(s)

        drain_gathers(1 - s)
        reduce_row(b - 1, 1 - s)
      return _

    lax.fori_loop(0, rows_per_w // 2, pair_body, 0, unroll=False)
    pltpu.sync_copy(out_v, out_hbm.at[pl.ds(base, rows_per_w)])

  return k


def kernel(x, mapping, mem):
  perm = mapping.reshape(N_NODES, NBITS).T.reshape(-1).astype(jnp.int32)
  mem2 = mem.reshape(N_NODES * N_ENTRIES, OUT_DIM)
  info = plsc.get_sparse_core_info()
  nw = info.num_cores * info.num_subcores
  k = _make_kernel(nw)
  return k(x, perm, mem2)
